# Initial kernel scaffold; baseline (speedup 1.0000x reference)
import jax
import jax.numpy as jnp
from jax.experimental import pallas as pl


def kernel(query_in_feat, data_in_feat, query_edge_list, data_edge_list, query2data_edge_list, qg_W1, qg_b1, qg_W2, qg_b2, qg_W3, qg_b3, qg_W4, qg_b4, dg_W1, dg_b1, dg_W2, dg_b2, dg_W3, dg_b3, dg_W4, dg_b4, gat_W, gat_a_src, gat_a_dst, gat_b, L1_W, L1_b, L2_W, L2_b, L3_W, L3_b, L4_W, L4_b):
    raise NotImplementedError("write your pallas kernel here")


# speedup vs baseline: 7.9463x; 7.9463x over previous
import functools

import jax
import jax.numpy as jnp
from jax import lax
from jax.experimental import pallas as pl
from jax.experimental.pallas import tpu as pltpu
from jax.experimental.pallas import tpu_sc as plsc

NC = 2
NS = 16
NW = NC * NS
L = 16

CH = 128
ZR = 128
F = 128


def _ceil_div(a, b):
    return (a + b - 1) // b


def _make_segsum(n_rows, n_acc, nchunks_per_tile):
    mesh = plsc.VectorSubcoreMesh(core_axis_name="c", subcore_axis_name="s")
    nzb = _ceil_div(n_acc, ZR)

    @functools.partial(
        pl.kernel,
        out_type=jax.ShapeDtypeStruct((NC, n_acc, F), jnp.float32),
        mesh=mesh,
        compiler_params=pltpu.CompilerParams(needs_layout_passes=False),
        scratch_types=[
            pltpu.VMEM((nchunks_per_tile, CH), jnp.int32),
            pltpu.VMEM((nchunks_per_tile, CH), jnp.int32),
            pltpu.VMEM((CH, F), jnp.float32),
            pltpu.VMEM_SHARED((n_rows, F), jnp.float32),
            pltpu.SemaphoreType.DMA,
        ],
    )
    def segsum(x_hbm, src_hbm, dst_hbm, out_hbm, sidx, didx, rows, acc, sem):
        core = lax.axis_index("c")
        sub = lax.axis_index("s")
        wid = sub * NC + core

        z16 = jnp.zeros((L,), jnp.float32)

        def zrow(i, _):
            for j in range(F // L):
                rows[i, pl.ds(j * L, L)] = z16
            return 0

        lax.fori_loop(0, CH, zrow, 0)

        def zblk(z, _):
            bz = sub + z * NS
            off = jnp.minimum(bz * ZR, n_acc - ZR)
            pltpu.sync_copy(rows, acc.at[pl.ds(off, ZR), :])
            return 0

        nz = (nzb - sub + NS - 1) // NS
        lax.fori_loop(0, nz, zblk, 0)
        if n_rows > n_acc:

            @pl.when(sub == 0)
            def _():
                pltpu.sync_copy(
                    rows.at[pl.ds(0, n_rows - n_acc), :],
                    acc.at[pl.ds(n_acc, n_rows - n_acc), :],
                )

        plsc.subcore_barrier()

        pltpu.sync_copy(src_hbm.at[wid], sidx)
        pltpu.sync_copy(dst_hbm.at[wid], didx)

        def chunk(k, _):
            pltpu.async_copy(x_hbm.at[sidx.at[k]], rows, sem).wait()
            pltpu.sync_copy(rows, acc.at[didx.at[k]], add=True)
            return 0

        lax.fori_loop(0, nchunks_per_tile, chunk, 0)

        plsc.subcore_barrier()

        def oblk(z, _):
            bz = sub + z * NS
            off = jnp.minimum(bz * ZR, n_acc - ZR)
            pltpu.sync_copy(
                acc.at[pl.ds(off, ZR), :], out_hbm.at[core, pl.ds(off, ZR), :]
            )
            return 0

        lax.fori_loop(0, nz, oblk, 0)

    return segsum


def _make_gat_edges(n_rows, n_acc, nchunks_per_tile):
    mesh = plsc.VectorSubcoreMesh(core_axis_name="c", subcore_axis_name="s")
    nzb = _ceil_div(n_acc, ZR)
    G8 = 8
    assert nchunks_per_tile % G8 == 0

    @functools.partial(
        pl.kernel,
        out_type=(
            jax.ShapeDtypeStruct((NC, n_acc, F), jnp.float32),
            jax.ShapeDtypeStruct((NC * n_acc,), jnp.float32),
        ),
        mesh=mesh,
        compiler_params=pltpu.CompilerParams(needs_layout_passes=False),
        scratch_types=[
            pltpu.VMEM((G8, CH), jnp.int32),
            pltpu.VMEM((G8, CH), jnp.int32),
            pltpu.VMEM((n_acc,), jnp.float32),
            pltpu.VMEM((n_acc,), jnp.float32),
            pltpu.VMEM((L,), jnp.float32),
            pltpu.VMEM((L,), jnp.float32),
            pltpu.VMEM((CH, F), jnp.float32),
            pltpu.VMEM((CH,), jnp.float32),
            pltpu.VMEM_SHARED((n_rows, F), jnp.float32),
            pltpu.VMEM_SHARED((n_rows,), jnp.float32),
            pltpu.SemaphoreType.DMA,
        ],
    )
    def gat(h_hbm, s_hbm, t_hbm, smax_hbm, tmax_hbm, src_hbm, dst_hbm,
            num_hbm, den_hbm, sidx, didx, sv, tv, smv, tmv, rows, ee,
            accn, accd, sem):
        core = lax.axis_index("c")
        sub = lax.axis_index("s")
        wid = sub * NC + core

        z16 = jnp.zeros((L,), jnp.float32)

        def zrow(i, _):
            for j in range(F // L):
                rows[i, pl.ds(j * L, L)] = z16
            return 0

        lax.fori_loop(0, CH, zrow, 0)
        for j in range(CH // L):
            ee[pl.ds(j * L, L)] = z16

        def zblk(z, _):
            bz = sub + z * NS
            off = jnp.minimum(bz * ZR, n_acc - ZR)
            pltpu.sync_copy(rows, accn.at[pl.ds(off, ZR), :])
            pltpu.sync_copy(ee, accd.at[pl.ds(off, CH)])
            return 0

        nz = (nzb - sub + NS - 1) // NS
        lax.fori_loop(0, nz, zblk, 0)

        if n_rows > n_acc:

            @pl.when(sub == 0)
            def _():
                pltpu.sync_copy(
                    rows.at[pl.ds(0, n_rows - n_acc), :],
                    accn.at[pl.ds(n_acc, n_rows - n_acc), :],
                )
                pltpu.sync_copy(
                    ee.at[pl.ds(0, n_rows - n_acc)],
                    accd.at[pl.ds(n_acc, n_rows - n_acc)],
                )

        pltpu.sync_copy(s_hbm, sv)
        pltpu.sync_copy(t_hbm, tv)
        pltpu.sync_copy(smax_hbm, smv)
        pltpu.sync_copy(tmax_hbm, tmv)

        plsc.subcore_barrier()

        cbound = jnp.maximum(smv[...] + tmv[...], 0.0)

        def group(g, _):
            pltpu.sync_copy(src_hbm.at[wid, pl.ds(g * G8, G8)], sidx)
            pltpu.sync_copy(dst_hbm.at[wid, pl.ds(g * G8, G8)], didx)

            def chunk(k, _):
                for j in range(CH // L):
                    si = sidx[k, pl.ds(j * L, L)]
                    di = didx[k, pl.ds(j * L, L)]
                    svv = plsc.load_gather(sv, [si])
                    tvv = plsc.load_gather(tv, [di])
                    e = svv + tvv
                    e = jnp.maximum(e, 0.2 * e)
                    ee[pl.ds(j * L, L)] = jnp.exp(e - cbound)

                pltpu.async_copy(h_hbm.at[sidx.at[k]], rows, sem).wait()

                def scale(i, _):
                    w = plsc.load_gather(ee, [jnp.full((L,), 0, jnp.int32) + i])
                    for j in range(F // L):
                        rows[i, pl.ds(j * L, L)] = rows[i, pl.ds(j * L, L)] * w
                    return 0

                lax.fori_loop(0, CH, scale, 0)

                pltpu.sync_copy(rows, accn.at[didx.at[k]], add=True)
                pltpu.sync_copy(ee, accd.at[didx.at[k]], add=True)
                return 0

            lax.fori_loop(0, G8, chunk, 0)
            return 0

        lax.fori_loop(0, nchunks_per_tile // G8, group, 0)

        plsc.subcore_barrier()

        def oblk(z, _):
            bz = sub + z * NS
            off = jnp.minimum(bz * ZR, n_acc - ZR)
            pltpu.sync_copy(
                accn.at[pl.ds(off, ZR), :], num_hbm.at[core, pl.ds(off, ZR), :]
            )
            pltpu.sync_copy(accd.at[pl.ds(off, CH)], ee)
            pltpu.sync_copy(ee, den_hbm.at[pl.ds(core * n_acc + off, CH)])
            return 0

        lax.fori_loop(0, nz, oblk, 0)

    return gat


def _mlp2(x, agg, w1, b1, w2, b2, off_blocks, outer_relu, want_sum):
    n = x.shape[0]
    B = 512
    grid = _ceil_div(n, B)

    def body(x_ref, a_ref, w1_ref, b1_ref, w2_ref, b2_ref, o_ref, *rest):
        i = pl.program_id(0)
        a = a_ref[...]
        xa = x_ref[...] + a[0] + a[1]
        h = jnp.maximum(
            jnp.dot(xa, w1_ref[...], preferred_element_type=jnp.float32)
            + b1_ref[...][None, :],
            0.0,
        )
        h = (
            jnp.dot(h, w2_ref[...], preferred_element_type=jnp.float32)
            + b2_ref[...][None, :]
        )
        if outer_relu:
            h = jnp.maximum(h, 0.0)
        o_ref[...] = h
        if want_sum:
            s_ref = rest[0]
            rows = i * B + lax.broadcasted_iota(jnp.int32, (B, 1), 0)
            hm = jnp.where(rows < n, h, 0.0)

            @pl.when(i == 0)
            def _():
                s_ref[...] = jnp.zeros((1, F), jnp.float32)

            s_ref[...] += hm.sum(axis=0, keepdims=True)

    out_shape = [jax.ShapeDtypeStruct((n, F), jnp.float32)]
    out_specs = [pl.BlockSpec((B, F), lambda i: (i, 0))]
    if want_sum:
        out_shape.append(jax.ShapeDtypeStruct((1, F), jnp.float32))
        out_specs.append(pl.BlockSpec((1, F), lambda i: (0, 0)))
    res = pl.pallas_call(
        body,
        grid=(grid,),
        in_specs=[
            pl.BlockSpec((B, F), lambda i: (i, 0)),
            pl.BlockSpec((NC, B, F), lambda i: (0, i + off_blocks, 0)),
            pl.BlockSpec((F, F), lambda i: (0, 0)),
            pl.BlockSpec((F,), lambda i: (0,)),
            pl.BlockSpec((F, F), lambda i: (0, 0)),
            pl.BlockSpec((F,), lambda i: (0,)),
        ],
        out_specs=out_specs,
        out_shape=out_shape,
    )(x, agg, w1, b1, w2, b2)
    return res if want_sum else res[0]


def _gat_pre(x, w, a_src, a_dst):
    n = x.shape[0]
    B = 1024
    grid = _ceil_div(n, B)
    neg = -3.0e38

    def body(x_ref, w_ref, as_ref, ad_ref, h_ref, s_ref, t_ref, sm_ref, tm_ref):
        i = pl.program_id(0)
        h = jnp.dot(x_ref[...], w_ref[...], preferred_element_type=jnp.float32)
        h_ref[...] = h
        s = jnp.dot(h, as_ref[...][:, None], preferred_element_type=jnp.float32)
        t = jnp.dot(h, ad_ref[...][:, None], preferred_element_type=jnp.float32)
        s_ref[...] = s
        t_ref[...] = t
        rows = i * B + lax.broadcasted_iota(jnp.int32, (B, 1), 0)
        valid = rows < n
        sm = jnp.max(jnp.where(valid, s, neg))
        tm = jnp.max(jnp.where(valid, t, neg))

        @pl.when(i == 0)
        def _():
            sm_ref[...] = jnp.full((L,), neg, jnp.float32)
            tm_ref[...] = jnp.full((L,), neg, jnp.float32)

        sm_ref[...] = jnp.maximum(sm_ref[...], sm)
        tm_ref[...] = jnp.maximum(tm_ref[...], tm)

    return pl.pallas_call(
        body,
        grid=(grid,),
        in_specs=[
            pl.BlockSpec((B, F), lambda i: (i, 0)),
            pl.BlockSpec((F, F), lambda i: (0, 0)),
            pl.BlockSpec((F,), lambda i: (0,)),
            pl.BlockSpec((F,), lambda i: (0,)),
        ],
        out_specs=[
            pl.BlockSpec((B, F), lambda i: (i, 0)),
            pl.BlockSpec((B, 1), lambda i: (i, 0)),
            pl.BlockSpec((B, 1), lambda i: (i, 0)),
            pl.BlockSpec((L,), lambda i: (0,)),
            pl.BlockSpec((L,), lambda i: (0,)),
        ],
        out_shape=[
            jax.ShapeDtypeStruct((n, F), jnp.float32),
            jax.ShapeDtypeStruct((n, 1), jnp.float32),
            jax.ShapeDtypeStruct((n, 1), jnp.float32),
            jax.ShapeDtypeStruct((L,), jnp.float32),
            jax.ShapeDtypeStruct((L,), jnp.float32),
        ],
    )(x, w, a_src, a_dst)


def _finalize(nump, denp, b, nq, ntot):
    B = 1024
    grid = _ceil_div(ntot, B)

    def body(n_ref, d_ref, b_ref, att_ref, qs_ref, ds_ref):
        i = pl.program_id(0)
        nsum = n_ref[...][0] + n_ref[...][1]
        den = d_ref[...][0] + d_ref[...][1] + 1e-16
        att = nsum / den[:, None] + b_ref[...][None, :]
        att_ref[...] = att
        rows = i * B + lax.broadcasted_iota(jnp.int32, (B, 1), 0)
        attv = jnp.where(rows < ntot, att, 0.0)
        qm = rows < nq

        @pl.when(i == 0)
        def _():
            qs_ref[...] = jnp.zeros((1, F), jnp.float32)
            ds_ref[...] = jnp.zeros((1, F), jnp.float32)

        qs_ref[...] += jnp.where(qm, attv, 0.0).sum(axis=0, keepdims=True)
        ds_ref[...] += jnp.where(qm, 0.0, attv).sum(axis=0, keepdims=True)

    return pl.pallas_call(
        body,
        grid=(grid,),
        in_specs=[
            pl.BlockSpec((NC, B, F), lambda i: (0, i, 0)),
            pl.BlockSpec((NC, B), lambda i: (0, i)),
            pl.BlockSpec((F,), lambda i: (0,)),
        ],
        out_specs=[
            pl.BlockSpec((B, F), lambda i: (i, 0)),
            pl.BlockSpec((1, F), lambda i: (0, 0)),
            pl.BlockSpec((1, F), lambda i: (0, 0)),
        ],
        out_shape=[
            jax.ShapeDtypeStruct((ntot, F), jnp.float32),
            jax.ShapeDtypeStruct((1, F), jnp.float32),
            jax.ShapeDtypeStruct((1, F), jnp.float32),
        ],
    )(nump, denp, b)


def _head(qa, qb, da, db, w1, b1, w2, b2, w3, b3, w4, b4):
    def body(qa_ref, qb_ref, da_ref, db_ref, w1_ref, b1_ref, w2_ref, b2_ref,
             w3_ref, b3_ref, w4_ref, b4_ref, o_ref):
        w1v = w1_ref[...]
        h = (
            jnp.dot(qa_ref[...], w1v[0:128], preferred_element_type=jnp.float32)
            + jnp.dot(qb_ref[...], w1v[128:256], preferred_element_type=jnp.float32)
            + jnp.dot(da_ref[...], w1v[256:384], preferred_element_type=jnp.float32)
            + jnp.dot(db_ref[...], w1v[384:512], preferred_element_type=jnp.float32)
            + b1_ref[...][None, :]
        )
        h = jnp.dot(h, w2_ref[...], preferred_element_type=jnp.float32) + b2_ref[...][None, :]
        h = jnp.maximum(h, 0.0)
        h = jnp.dot(h, w3_ref[...], preferred_element_type=jnp.float32) + b3_ref[...][None, :]
        h = jnp.maximum(h, 0.0)
        h = jnp.dot(h, w4_ref[...], preferred_element_type=jnp.float32) + b4_ref[...][None, :]
        o_ref[...] = jnp.maximum(h, 0.0)

    return pl.pallas_call(
        body,
        out_shape=jax.ShapeDtypeStruct((1, 1), jnp.float32),
    )(qa, qb, da, db, w1, b1, w2, b2, w3, b3, w4, b4)


def _prep_edges(src, dst, n_acc, n_rows):
    e = src.shape[0]
    unit = CH * NW
    epad = _ceil_div(e, unit) * unit
    npad = epad - e
    if npad:
        fill_src = (jnp.arange(npad, dtype=jnp.int32) % 64)
        fill_dst = n_acc + (jnp.arange(npad, dtype=jnp.int32) % (n_rows - n_acc))
        src = jnp.concatenate([src, fill_src])
        dst = jnp.concatenate([dst, fill_dst])
    per = epad // (NW * CH)
    src3 = src.reshape(NW, per, CH)
    dst3 = dst.reshape(NW, per, CH)
    return src3, dst3, per


def kernel(query_in_feat, data_in_feat, query_edge_list, data_edge_list,
           query2data_edge_list, qg_W1, qg_b1, qg_W2, qg_b2, qg_W3, qg_b3,
           qg_W4, qg_b4, dg_W1, dg_b1, dg_W2, dg_b2, dg_W3, dg_b3, dg_W4,
           dg_b4, gat_W, gat_a_src, gat_a_dst, gat_b, L1_W, L1_b, L2_W, L2_b,
           L3_W, L3_b, L4_W, L4_b):
    nq = query_in_feat.shape[0]
    nd = data_in_feat.shape[0]
    ntot = nq + nd
    n_rows = ntot + 8

    qe = query_edge_list.astype(jnp.int32)
    de = data_edge_list.astype(jnp.int32)
    xe = query2data_edge_list.astype(jnp.int32)

    csrc = jnp.concatenate([qe[0], de[0] + nq])
    cdst = jnp.concatenate([qe[1], de[1] + nq])
    csrc3, cdst3, cper = _prep_edges(csrc, cdst, ntot, n_rows)
    xsrc3, xdst3, xper = _prep_edges(xe[0], xe[1], ntot, n_rows)

    segsum = _make_segsum(n_rows, ntot, cper)
    gat_edges = _make_gat_edges(n_rows, ntot, xper)

    x0 = jnp.concatenate([query_in_feat, data_in_feat], axis=0)

    agg1 = segsum(x0, csrc3, cdst3)
    hq = _mlp2(query_in_feat, agg1, qg_W1, qg_b1, qg_W2, qg_b2,
               off_blocks=0, outer_relu=True, want_sum=False)
    hd = _mlp2(data_in_feat, agg1, dg_W1, dg_b1, dg_W2, dg_b2,
               off_blocks=1, outer_relu=True, want_sum=False)

    x1 = jnp.concatenate([hq, hd], axis=0)
    agg2 = segsum(x1, csrc3, cdst3)
    query_x, qsA = _mlp2(hq, agg2, qg_W3, qg_b3, qg_W4, qg_b4,
                         off_blocks=0, outer_relu=False, want_sum=True)
    data_x, dsA = _mlp2(hd, agg2, dg_W3, dg_b3, dg_W4, dg_b4,
                        off_blocks=1, outer_relu=False, want_sum=True)

    hg, s, t, smax, tmax = _gat_pre(x0, gat_W, gat_a_src, gat_a_dst)
    nump, denp = gat_edges(hg, s[:, 0], t[:, 0], smax, tmax, xsrc3, xdst3)
    att, qsB, dsB = _finalize(nump, denp.reshape(NC, ntot), gat_b, nq, ntot)

    pred = _head(qsA, qsB, dsA, dsB, L1_W, L1_b, L2_W, L2_b, L3_W, L3_b,
                 L4_W, L4_b)

    out_q = jnp.concatenate([query_x, att[:nq]], axis=1)
    out_d = jnp.concatenate([data_x, att[nq:]], axis=1)
    return (pred, out_q, out_d)


# speedup vs baseline: 9.3971x; 1.1826x over previous
import functools

import jax
import jax.numpy as jnp
from jax import lax
from jax.experimental import pallas as pl
from jax.experimental.pallas import tpu as pltpu
from jax.experimental.pallas import tpu_sc as plsc

NC = 2
NS = 16
NW = NC * NS
L = 16

CH = 64
ZR = 128
F = 128


def _ceil_div(a, b):
    return (a + b - 1) // b


G = 16


def _make_segsum(n_rows, n_acc, nchunks_per_tile):
    mesh = plsc.VectorSubcoreMesh(core_axis_name="c", subcore_axis_name="s")
    nzb = _ceil_div(n_acc, CH)
    nob = _ceil_div(n_acc, ZR)
    assert nchunks_per_tile % G == 0

    @functools.partial(
        pl.kernel,
        out_type=jax.ShapeDtypeStruct((NC, n_acc, F), jnp.float32),
        mesh=mesh,
        compiler_params=pltpu.CompilerParams(needs_layout_passes=False),
        scratch_types=[
            pltpu.VMEM((G, 2, CH), jnp.int32),
            pltpu.VMEM((2, CH, F), jnp.float32),
            pltpu.VMEM_SHARED((n_rows, F), jnp.float32),
            pltpu.SemaphoreType.DMA,
            pltpu.SemaphoreType.DMA,
        ],
    )
    def segsum(x_hbm, eidx_hbm, out_hbm, ibuf, rows, acc, gs0, gs1):
        core = lax.axis_index("c")
        sub = lax.axis_index("s")
        wid = sub * NC + core

        z16 = jnp.zeros((L,), jnp.float32)

        def zrow(i, _):
            for j in range(F // L):
                rows[0, i, pl.ds(j * L, L)] = z16
            return 0

        lax.fori_loop(0, CH, zrow, 0)

        def zblk(z, _):
            bz = sub + z * NS
            off = jnp.minimum(bz * CH, n_acc - CH)
            pltpu.sync_copy(rows.at[0], acc.at[pl.ds(off, CH), :])
            return 0

        nz = (nzb - sub + NS - 1) // NS
        lax.fori_loop(0, nz, zblk, 0)
        if n_rows > n_acc:

            @pl.when(sub == 0)
            def _():
                pltpu.sync_copy(
                    rows.at[0, pl.ds(0, n_rows - n_acc), :],
                    acc.at[pl.ds(n_acc, n_rows - n_acc), :],
                )

        plsc.subcore_barrier()

        def gather(kk, b, sem):
            return pltpu.make_async_copy(
                x_hbm.at[ibuf.at[kk, 0]], rows.at[b], sem
            )

        def scat(kk, b):
            pltpu.sync_copy(rows.at[b], acc.at[ibuf.at[kk, 1]], add=True)

        def grp(g, _):
            pltpu.sync_copy(eidx_hbm.at[wid, pl.ds(g * G, G)], ibuf)
            gather(0, 0, gs0).start()

            def pair(p, _):
                k0 = 2 * p
                k1 = k0 + 1
                gather(k1, 1, gs1).start()
                gather(k0, 0, gs0).wait()
                scat(k0, 0)

                @pl.when(k1 + 1 < G)
                def _():
                    gather(k1 + 1, 0, gs0).start()

                gather(k1, 1, gs1).wait()
                scat(k1, 1)
                return 0

            lax.fori_loop(0, G // 2, pair, 0)
            return 0

        lax.fori_loop(0, nchunks_per_tile // G, grp, 0)

        plsc.subcore_barrier()

        def oblk(z, _):
            bz = sub + z * NS
            off = jnp.minimum(bz * ZR, n_acc - ZR)
            pltpu.sync_copy(
                acc.at[pl.ds(off, ZR), :], out_hbm.at[core, pl.ds(off, ZR), :]
            )
            return 0

        no = (nob - sub + NS - 1) // NS
        lax.fori_loop(0, no, oblk, 0)

    return segsum


def _make_gat_edges(n_rows, n_acc, nchunks_per_tile):
    mesh = plsc.VectorSubcoreMesh(core_axis_name="c", subcore_axis_name="s")
    nzb = _ceil_div(n_acc, CH)
    nob = _ceil_div(n_acc, ZR)
    assert nchunks_per_tile % G == 0

    @functools.partial(
        pl.kernel,
        out_type=(
            jax.ShapeDtypeStruct((NC, n_acc, F), jnp.float32),
            jax.ShapeDtypeStruct((NC * n_acc,), jnp.float32),
        ),
        mesh=mesh,
        compiler_params=pltpu.CompilerParams(needs_layout_passes=False),
        scratch_types=[
            pltpu.VMEM((G, 2, CH), jnp.int32),
            pltpu.VMEM((n_acc,), jnp.float32),
            pltpu.VMEM((n_acc,), jnp.float32),
            pltpu.VMEM((L,), jnp.float32),
            pltpu.VMEM((L,), jnp.float32),
            pltpu.VMEM((2, CH, F), jnp.float32),
            pltpu.VMEM((CH,), jnp.float32),
            pltpu.VMEM_SHARED((n_rows, F), jnp.float32),
            pltpu.VMEM_SHARED((n_rows,), jnp.float32),
            pltpu.SemaphoreType.DMA,
            pltpu.SemaphoreType.DMA,
        ],
    )
    def gat(h_hbm, s_hbm, t_hbm, smax_hbm, tmax_hbm, eidx_hbm,
            num_hbm, den_hbm, ibuf, sv, tv, smv, tmv, rows, ee,
            accn, accd, gs0, gs1):
        core = lax.axis_index("c")
        sub = lax.axis_index("s")
        wid = sub * NC + core

        z16 = jnp.zeros((L,), jnp.float32)

        def zrow(i, _):
            for j in range(F // L):
                rows[0, i, pl.ds(j * L, L)] = z16
            return 0

        lax.fori_loop(0, CH, zrow, 0)
        for j in range(CH // L):
            ee[pl.ds(j * L, L)] = z16

        def zblk(z, _):
            bz = sub + z * NS
            off = jnp.minimum(bz * CH, n_acc - CH)
            pltpu.sync_copy(rows.at[0], accn.at[pl.ds(off, CH), :])
            pltpu.sync_copy(ee, accd.at[pl.ds(off, CH)])
            return 0

        nz = (nzb - sub + NS - 1) // NS
        lax.fori_loop(0, nz, zblk, 0)

        if n_rows > n_acc:

            @pl.when(sub == 0)
            def _():
                pltpu.sync_copy(
                    rows.at[0, pl.ds(0, n_rows - n_acc), :],
                    accn.at[pl.ds(n_acc, n_rows - n_acc), :],
                )
                pltpu.sync_copy(
                    ee.at[pl.ds(0, n_rows - n_acc)],
                    accd.at[pl.ds(n_acc, n_rows - n_acc)],
                )

        pltpu.sync_copy(s_hbm, sv)
        pltpu.sync_copy(t_hbm, tv)
        pltpu.sync_copy(smax_hbm, smv)
        pltpu.sync_copy(tmax_hbm, tmv)

        plsc.subcore_barrier()

        cbound = jnp.maximum(smv[...] + tmv[...], 0.0)

        def gather(kk, b, sem):
            return pltpu.make_async_copy(
                h_hbm.at[ibuf.at[kk, 0]], rows.at[b], sem
            )

        def process(kk, b):
            for j in range(CH // L):
                si = ibuf[kk, 0, pl.ds(j * L, L)]
                di = ibuf[kk, 1, pl.ds(j * L, L)]
                svv = plsc.load_gather(sv, [si])
                tvv = plsc.load_gather(tv, [di])
                e = svv + tvv
                e = jnp.maximum(e, 0.2 * e)
                ee[pl.ds(j * L, L)] = jnp.exp(e - cbound)

            def scale(i, _):
                w = plsc.load_gather(ee, [jnp.full((L,), 0, jnp.int32) + i])
                for j in range(F // L):
                    rows[b, i, pl.ds(j * L, L)] = rows[b, i, pl.ds(j * L, L)] * w
                return 0

            lax.fori_loop(0, CH, scale, 0)
            pltpu.sync_copy(rows.at[b], accn.at[ibuf.at[kk, 1]], add=True)
            pltpu.sync_copy(ee, accd.at[ibuf.at[kk, 1]], add=True)

        def grp(g, _):
            pltpu.sync_copy(eidx_hbm.at[wid, pl.ds(g * G, G)], ibuf)
            gather(0, 0, gs0).start()

            def pair(p, _):
                k0 = 2 * p
                k1 = k0 + 1
                gather(k1, 1, gs1).start()
                gather(k0, 0, gs0).wait()
                process(k0, 0)

                @pl.when(k1 + 1 < G)
                def _():
                    gather(k1 + 1, 0, gs0).start()

                gather(k1, 1, gs1).wait()
                process(k1, 1)
                return 0

            lax.fori_loop(0, G // 2, pair, 0)
            return 0

        lax.fori_loop(0, nchunks_per_tile // G, grp, 0)

        plsc.subcore_barrier()

        def oblk(z, _):
            bz = sub + z * NS
            off = jnp.minimum(bz * ZR, n_acc - ZR)
            pltpu.sync_copy(
                accn.at[pl.ds(off, ZR), :], num_hbm.at[core, pl.ds(off, ZR), :]
            )
            return 0

        no = (nob - sub + NS - 1) // NS
        lax.fori_loop(0, no, oblk, 0)

        def oblkd(z, _):
            bz = sub + z * NS
            off = jnp.minimum(bz * CH, n_acc - CH)
            pltpu.sync_copy(accd.at[pl.ds(off, CH)], ee)
            pltpu.sync_copy(ee, den_hbm.at[pl.ds(core * n_acc + off, CH)])
            return 0

        lax.fori_loop(0, nz, oblkd, 0)

    return gat


def _mlp2(x, agg, w1, b1, w2, b2, off_blocks, outer_relu, want_sum):
    n = x.shape[0]
    B = 512
    grid = _ceil_div(n, B)

    def body(x_ref, a_ref, w1_ref, b1_ref, w2_ref, b2_ref, o_ref, *rest):
        i = pl.program_id(0)
        a = a_ref[...]
        xa = x_ref[...] + a[0] + a[1]
        h = jnp.maximum(
            jnp.dot(xa, w1_ref[...], preferred_element_type=jnp.float32)
            + b1_ref[...][None, :],
            0.0,
        )
        h = (
            jnp.dot(h, w2_ref[...], preferred_element_type=jnp.float32)
            + b2_ref[...][None, :]
        )
        if outer_relu:
            h = jnp.maximum(h, 0.0)
        o_ref[...] = h
        if want_sum:
            s_ref = rest[0]
            rows = i * B + lax.broadcasted_iota(jnp.int32, (B, 1), 0)
            hm = jnp.where(rows < n, h, 0.0)

            @pl.when(i == 0)
            def _():
                s_ref[...] = jnp.zeros((1, F), jnp.float32)

            s_ref[...] += hm.sum(axis=0, keepdims=True)

    out_shape = [jax.ShapeDtypeStruct((n, F), jnp.float32)]
    out_specs = [pl.BlockSpec((B, F), lambda i: (i, 0))]
    if want_sum:
        out_shape.append(jax.ShapeDtypeStruct((1, F), jnp.float32))
        out_specs.append(pl.BlockSpec((1, F), lambda i: (0, 0)))
    res = pl.pallas_call(
        body,
        grid=(grid,),
        in_specs=[
            pl.BlockSpec((B, F), lambda i: (i, 0)),
            pl.BlockSpec((NC, B, F), lambda i: (0, i + off_blocks, 0)),
            pl.BlockSpec((F, F), lambda i: (0, 0)),
            pl.BlockSpec((F,), lambda i: (0,)),
            pl.BlockSpec((F, F), lambda i: (0, 0)),
            pl.BlockSpec((F,), lambda i: (0,)),
        ],
        out_specs=out_specs,
        out_shape=out_shape,
    )(x, agg, w1, b1, w2, b2)
    return res if want_sum else res[0]


def _gat_pre(x, w, a_src, a_dst):
    n = x.shape[0]
    B = 1024
    grid = _ceil_div(n, B)
    neg = -3.0e38

    def body(x_ref, w_ref, as_ref, ad_ref, h_ref, s_ref, t_ref, sm_ref, tm_ref):
        i = pl.program_id(0)
        h = jnp.dot(x_ref[...], w_ref[...], preferred_element_type=jnp.float32)
        h_ref[...] = h
        s = jnp.dot(h, as_ref[...][:, None], preferred_element_type=jnp.float32)
        t = jnp.dot(h, ad_ref[...][:, None], preferred_element_type=jnp.float32)
        s_ref[...] = s
        t_ref[...] = t
        rows = i * B + lax.broadcasted_iota(jnp.int32, (B, 1), 0)
        valid = rows < n
        sm = jnp.max(jnp.where(valid, s, neg))
        tm = jnp.max(jnp.where(valid, t, neg))

        @pl.when(i == 0)
        def _():
            sm_ref[...] = jnp.full((L,), neg, jnp.float32)
            tm_ref[...] = jnp.full((L,), neg, jnp.float32)

        sm_ref[...] = jnp.maximum(sm_ref[...], sm)
        tm_ref[...] = jnp.maximum(tm_ref[...], tm)

    return pl.pallas_call(
        body,
        grid=(grid,),
        in_specs=[
            pl.BlockSpec((B, F), lambda i: (i, 0)),
            pl.BlockSpec((F, F), lambda i: (0, 0)),
            pl.BlockSpec((F,), lambda i: (0,)),
            pl.BlockSpec((F,), lambda i: (0,)),
        ],
        out_specs=[
            pl.BlockSpec((B, F), lambda i: (i, 0)),
            pl.BlockSpec((B, 1), lambda i: (i, 0)),
            pl.BlockSpec((B, 1), lambda i: (i, 0)),
            pl.BlockSpec((L,), lambda i: (0,)),
            pl.BlockSpec((L,), lambda i: (0,)),
        ],
        out_shape=[
            jax.ShapeDtypeStruct((n, F), jnp.float32),
            jax.ShapeDtypeStruct((n, 1), jnp.float32),
            jax.ShapeDtypeStruct((n, 1), jnp.float32),
            jax.ShapeDtypeStruct((L,), jnp.float32),
            jax.ShapeDtypeStruct((L,), jnp.float32),
        ],
    )(x, w, a_src, a_dst)


def _finalize(nump, denp, b, nq, ntot):
    B = 1024
    grid = _ceil_div(ntot, B)

    def body(n_ref, d_ref, b_ref, att_ref, qs_ref, ds_ref):
        i = pl.program_id(0)
        nsum = n_ref[...][0] + n_ref[...][1]
        den = d_ref[...][0] + d_ref[...][1] + 1e-16
        att = nsum / den[:, None] + b_ref[...][None, :]
        att_ref[...] = att
        rows = i * B + lax.broadcasted_iota(jnp.int32, (B, 1), 0)
        attv = jnp.where(rows < ntot, att, 0.0)
        qm = rows < nq

        @pl.when(i == 0)
        def _():
            qs_ref[...] = jnp.zeros((1, F), jnp.float32)
            ds_ref[...] = jnp.zeros((1, F), jnp.float32)

        qs_ref[...] += jnp.where(qm, attv, 0.0).sum(axis=0, keepdims=True)
        ds_ref[...] += jnp.where(qm, 0.0, attv).sum(axis=0, keepdims=True)

    return pl.pallas_call(
        body,
        grid=(grid,),
        in_specs=[
            pl.BlockSpec((NC, B, F), lambda i: (0, i, 0)),
            pl.BlockSpec((NC, B), lambda i: (0, i)),
            pl.BlockSpec((F,), lambda i: (0,)),
        ],
        out_specs=[
            pl.BlockSpec((B, F), lambda i: (i, 0)),
            pl.BlockSpec((1, F), lambda i: (0, 0)),
            pl.BlockSpec((1, F), lambda i: (0, 0)),
        ],
        out_shape=[
            jax.ShapeDtypeStruct((ntot, F), jnp.float32),
            jax.ShapeDtypeStruct((1, F), jnp.float32),
            jax.ShapeDtypeStruct((1, F), jnp.float32),
        ],
    )(nump, denp, b)


def _head(qa, qb, da, db, w1, b1, w2, b2, w3, b3, w4, b4):
    def body(qa_ref, qb_ref, da_ref, db_ref, w1_ref, b1_ref, w2_ref, b2_ref,
             w3_ref, b3_ref, w4_ref, b4_ref, o_ref):
        w1v = w1_ref[...]
        h = (
            jnp.dot(qa_ref[...], w1v[0:128], preferred_element_type=jnp.float32)
            + jnp.dot(qb_ref[...], w1v[128:256], preferred_element_type=jnp.float32)
            + jnp.dot(da_ref[...], w1v[256:384], preferred_element_type=jnp.float32)
            + jnp.dot(db_ref[...], w1v[384:512], preferred_element_type=jnp.float32)
            + b1_ref[...][None, :]
        )
        h = jnp.dot(h, w2_ref[...], preferred_element_type=jnp.float32) + b2_ref[...][None, :]
        h = jnp.maximum(h, 0.0)
        h = jnp.dot(h, w3_ref[...], preferred_element_type=jnp.float32) + b3_ref[...][None, :]
        h = jnp.maximum(h, 0.0)
        h = jnp.dot(h, w4_ref[...], preferred_element_type=jnp.float32) + b4_ref[...][None, :]
        o_ref[...] = jnp.maximum(h, 0.0)

    return pl.pallas_call(
        body,
        out_shape=jax.ShapeDtypeStruct((1, 1), jnp.float32),
    )(qa, qb, da, db, w1, b1, w2, b2, w3, b3, w4, b4)


def _prep_edges(src, dst, n_acc):
    e = src.shape[0]
    unit = 2 * CH * NW
    epad = _ceil_div(e, unit) * unit
    npad = epad - e
    if npad:
        fill_src = (jnp.arange(npad, dtype=jnp.int32) % 64)
        fill_dst = n_acc + (jnp.arange(npad, dtype=jnp.int32) % 8)
        src = jnp.concatenate([src, fill_src])
        dst = jnp.concatenate([dst, fill_dst])
    per = epad // (NW * CH)
    src3 = src.reshape(NW, per, CH)
    dst3 = dst.reshape(NW, per, CH)
    eidx = jnp.stack([src3, dst3], axis=2)
    return eidx, per, npad


def kernel(query_in_feat, data_in_feat, query_edge_list, data_edge_list,
           query2data_edge_list, qg_W1, qg_b1, qg_W2, qg_b2, qg_W3, qg_b3,
           qg_W4, qg_b4, dg_W1, dg_b1, dg_W2, dg_b2, dg_W3, dg_b3, dg_W4,
           dg_b4, gat_W, gat_a_src, gat_a_dst, gat_b, L1_W, L1_b, L2_W, L2_b,
           L3_W, L3_b, L4_W, L4_b):
    nq = query_in_feat.shape[0]
    nd = data_in_feat.shape[0]
    ntot = nq + nd

    qe = query_edge_list.astype(jnp.int32)
    de = data_edge_list.astype(jnp.int32)
    xe = query2data_edge_list.astype(jnp.int32)

    csrc = jnp.concatenate([qe[0], de[0] + nq])
    cdst = jnp.concatenate([qe[1], de[1] + nq])
    cidx, cper, cpad = _prep_edges(csrc, cdst, ntot)
    xidx, xper, xpad = _prep_edges(xe[0], xe[1], ntot)

    segsum = _make_segsum(ntot + (8 if cpad else 0), ntot, cper)
    gat_edges = _make_gat_edges(ntot + (8 if xpad else 0), ntot, xper)

    x0 = jnp.concatenate([query_in_feat, data_in_feat], axis=0)

    agg1 = segsum(x0, cidx)
    hq = _mlp2(query_in_feat, agg1, qg_W1, qg_b1, qg_W2, qg_b2,
               off_blocks=0, outer_relu=True, want_sum=False)
    hd = _mlp2(data_in_feat, agg1, dg_W1, dg_b1, dg_W2, dg_b2,
               off_blocks=1, outer_relu=True, want_sum=False)

    x1 = jnp.concatenate([hq, hd], axis=0)
    agg2 = segsum(x1, cidx)
    query_x, qsA = _mlp2(hq, agg2, qg_W3, qg_b3, qg_W4, qg_b4,
                         off_blocks=0, outer_relu=False, want_sum=True)
    data_x, dsA = _mlp2(hd, agg2, dg_W3, dg_b3, dg_W4, dg_b4,
                        off_blocks=1, outer_relu=False, want_sum=True)

    hg, s, t, smax, tmax = _gat_pre(x0, gat_W, gat_a_src, gat_a_dst)
    nump, denp = gat_edges(hg, s[:, 0], t[:, 0], smax, tmax, xidx)
    att, qsB, dsB = _finalize(nump, denp.reshape(NC, ntot), gat_b, nq, ntot)

    pred = _head(qsA, qsB, dsA, dsB, L1_W, L1_b, L2_W, L2_b, L3_W, L3_b,
                 L4_W, L4_b)

    out_q = jnp.concatenate([query_x, att[:nq]], axis=1)
    out_d = jnp.concatenate([data_x, att[nq:]], axis=1)
    return (pred, out_q, out_d)


# speedup vs baseline: 10.8172x; 1.1511x over previous
import functools

import jax
import jax.numpy as jnp
from jax import lax
from jax.experimental import pallas as pl
from jax.experimental.pallas import tpu as pltpu
from jax.experimental.pallas import tpu_sc as plsc

NC = 2
NS = 16
NW = NC * NS
L = 16

CH_GIN = 128
CH_GAT = 64
ZR = 128
F = 128


def _ceil_div(a, b):
    return (a + b - 1) // b


G = 16


def _make_segsum(n_rows, n_acc, nchunks_per_tile, ch):
    mesh = plsc.VectorSubcoreMesh(core_axis_name="c", subcore_axis_name="s")
    nzb = _ceil_div(n_acc, ch)
    nob = _ceil_div(n_acc, ZR)
    assert nchunks_per_tile % G == 0

    @functools.partial(
        pl.kernel,
        out_type=jax.ShapeDtypeStruct((NC, n_acc, F), jnp.float32),
        mesh=mesh,
        compiler_params=pltpu.CompilerParams(needs_layout_passes=False),
        scratch_types=[
            pltpu.VMEM((G, 2, ch), jnp.int32),
            pltpu.VMEM((2, ch, F), jnp.float32),
            pltpu.VMEM_SHARED((n_rows, F), jnp.float32),
            pltpu.SemaphoreType.DMA,
            pltpu.SemaphoreType.DMA,
        ],
    )
    def segsum(x_hbm, eidx_hbm, out_hbm, ibuf, rows, acc, gs0, gs1):
        core = lax.axis_index("c")
        sub = lax.axis_index("s")
        wid = sub * NC + core

        z16 = jnp.zeros((L,), jnp.float32)

        def zrow(i, _):
            for j in range(F // L):
                rows[0, i, pl.ds(j * L, L)] = z16
            return 0

        lax.fori_loop(0, ch, zrow, 0)

        def zblk(z, _):
            bz = sub + z * NS
            off = jnp.minimum(bz * ch, n_acc - ch)
            pltpu.sync_copy(rows.at[0], acc.at[pl.ds(off, ch), :])
            return 0

        nz = (nzb - sub + NS - 1) // NS
        lax.fori_loop(0, nz, zblk, 0)
        if n_rows > n_acc:

            @pl.when(sub == 0)
            def _():
                pltpu.sync_copy(
                    rows.at[0, pl.ds(0, n_rows - n_acc), :],
                    acc.at[pl.ds(n_acc, n_rows - n_acc), :],
                )

        plsc.subcore_barrier()

        def gather(kk, b, sem):
            return pltpu.make_async_copy(
                x_hbm.at[ibuf.at[kk, 0]], rows.at[b], sem
            )

        def scat(kk, b):
            pltpu.sync_copy(rows.at[b], acc.at[ibuf.at[kk, 1]], add=True)

        def grp(g, _):
            pltpu.sync_copy(eidx_hbm.at[wid, pl.ds(g * G, G)], ibuf)
            gather(0, 0, gs0).start()

            def pair(p, _):
                k0 = 2 * p
                k1 = k0 + 1
                gather(k1, 1, gs1).start()
                gather(k0, 0, gs0).wait()
                scat(k0, 0)

                @pl.when(k1 + 1 < G)
                def _():
                    gather(k1 + 1, 0, gs0).start()

                gather(k1, 1, gs1).wait()
                scat(k1, 1)
                return 0

            lax.fori_loop(0, G // 2, pair, 0)
            return 0

        lax.fori_loop(0, nchunks_per_tile // G, grp, 0)

        plsc.subcore_barrier()

        def oblk(z, _):
            bz = sub + z * NS
            off = jnp.minimum(bz * ZR, n_acc - ZR)
            pltpu.sync_copy(
                acc.at[pl.ds(off, ZR), :], out_hbm.at[core, pl.ds(off, ZR), :]
            )
            return 0

        no = (nob - sub + NS - 1) // NS
        lax.fori_loop(0, no, oblk, 0)

    return segsum


def _make_gat_edges(n_rows, n_acc, nchunks_per_tile, ch):
    mesh = plsc.VectorSubcoreMesh(core_axis_name="c", subcore_axis_name="s")
    nzb = _ceil_div(n_acc, ch)
    nob = _ceil_div(n_acc, ZR)
    assert nchunks_per_tile % G == 0

    @functools.partial(
        pl.kernel,
        out_type=(
            jax.ShapeDtypeStruct((NC, n_acc, F), jnp.float32),
            jax.ShapeDtypeStruct((NC * n_acc,), jnp.float32),
        ),
        mesh=mesh,
        compiler_params=pltpu.CompilerParams(needs_layout_passes=False),
        scratch_types=[
            pltpu.VMEM((G, 2, ch), jnp.int32),
            pltpu.VMEM((n_acc,), jnp.float32),
            pltpu.VMEM((n_acc,), jnp.float32),
            pltpu.VMEM((L,), jnp.float32),
            pltpu.VMEM((L,), jnp.float32),
            pltpu.VMEM((2, ch, F), jnp.float32),
            pltpu.VMEM((ch,), jnp.float32),
            pltpu.VMEM_SHARED((n_rows, F), jnp.float32),
            pltpu.VMEM_SHARED((n_rows,), jnp.float32),
            pltpu.SemaphoreType.DMA,
            pltpu.SemaphoreType.DMA,
        ],
    )
    def gat(h_hbm, s_hbm, t_hbm, smax_hbm, tmax_hbm, eidx_hbm,
            num_hbm, den_hbm, ibuf, sv, tv, smv, tmv, rows, ee,
            accn, accd, gs0, gs1):
        core = lax.axis_index("c")
        sub = lax.axis_index("s")
        wid = sub * NC + core

        z16 = jnp.zeros((L,), jnp.float32)

        def zrow(i, _):
            for j in range(F // L):
                rows[0, i, pl.ds(j * L, L)] = z16
            return 0

        lax.fori_loop(0, ch, zrow, 0)
        for j in range(ch // L):
            ee[pl.ds(j * L, L)] = z16

        def zblk(z, _):
            bz = sub + z * NS
            off = jnp.minimum(bz * ch, n_acc - ch)
            pltpu.sync_copy(rows.at[0], accn.at[pl.ds(off, ch), :])
            pltpu.sync_copy(ee, accd.at[pl.ds(off, ch)])
            return 0

        nz = (nzb - sub + NS - 1) // NS
        lax.fori_loop(0, nz, zblk, 0)

        if n_rows > n_acc:

            @pl.when(sub == 0)
            def _():
                pltpu.sync_copy(
                    rows.at[0, pl.ds(0, n_rows - n_acc), :],
                    accn.at[pl.ds(n_acc, n_rows - n_acc), :],
                )
                pltpu.sync_copy(
                    ee.at[pl.ds(0, n_rows - n_acc)],
                    accd.at[pl.ds(n_acc, n_rows - n_acc)],
                )

        pltpu.sync_copy(s_hbm, sv)
        pltpu.sync_copy(t_hbm, tv)
        pltpu.sync_copy(smax_hbm, smv)
        pltpu.sync_copy(tmax_hbm, tmv)

        plsc.subcore_barrier()

        cbound = jnp.maximum(smv[...] + tmv[...], 0.0)

        def gather(kk, b, sem):
            return pltpu.make_async_copy(
                h_hbm.at[ibuf.at[kk, 0]], rows.at[b], sem
            )

        def process(kk, b):
            for j in range(ch // L):
                si = ibuf[kk, 0, pl.ds(j * L, L)]
                di = ibuf[kk, 1, pl.ds(j * L, L)]
                svv = plsc.load_gather(sv, [si])
                tvv = plsc.load_gather(tv, [di])
                e = svv + tvv
                e = jnp.maximum(e, 0.2 * e)
                ee[pl.ds(j * L, L)] = jnp.exp(e - cbound)

            def scale(i, _):
                w = plsc.load_gather(ee, [jnp.full((L,), 0, jnp.int32) + i])
                for j in range(F // L):
                    rows[b, i, pl.ds(j * L, L)] = rows[b, i, pl.ds(j * L, L)] * w
                return 0

            lax.fori_loop(0, ch, scale, 0)
            pltpu.sync_copy(rows.at[b], accn.at[ibuf.at[kk, 1]], add=True)
            pltpu.sync_copy(ee, accd.at[ibuf.at[kk, 1]], add=True)

        def grp(g, _):
            pltpu.sync_copy(eidx_hbm.at[wid, pl.ds(g * G, G)], ibuf)
            gather(0, 0, gs0).start()

            def pair(p, _):
                k0 = 2 * p
                k1 = k0 + 1
                gather(k1, 1, gs1).start()
                gather(k0, 0, gs0).wait()
                process(k0, 0)

                @pl.when(k1 + 1 < G)
                def _():
                    gather(k1 + 1, 0, gs0).start()

                gather(k1, 1, gs1).wait()
                process(k1, 1)
                return 0

            lax.fori_loop(0, G // 2, pair, 0)
            return 0

        lax.fori_loop(0, nchunks_per_tile // G, grp, 0)

        plsc.subcore_barrier()

        def oblk(z, _):
            bz = sub + z * NS
            off = jnp.minimum(bz * ZR, n_acc - ZR)
            pltpu.sync_copy(
                accn.at[pl.ds(off, ZR), :], num_hbm.at[core, pl.ds(off, ZR), :]
            )
            return 0

        no = (nob - sub + NS - 1) // NS
        lax.fori_loop(0, no, oblk, 0)

        def oblkd(z, _):
            bz = sub + z * NS
            off = jnp.minimum(bz * ch, n_acc - ch)
            pltpu.sync_copy(accd.at[pl.ds(off, ch)], ee)
            pltpu.sync_copy(ee, den_hbm.at[pl.ds(core * n_acc + off, ch)])
            return 0

        lax.fori_loop(0, nz, oblkd, 0)

    return gat


def _mlp2(x, agg, w1, b1, w2, b2, off_blocks, outer_relu, want_sum):
    n = x.shape[0]
    B = 512
    grid = _ceil_div(n, B)

    def body(x_ref, a_ref, w1_ref, b1_ref, w2_ref, b2_ref, o_ref, *rest):
        i = pl.program_id(0)
        a = a_ref[...]
        xa = x_ref[...] + a[0] + a[1]
        h = jnp.maximum(
            jnp.dot(xa, w1_ref[...], preferred_element_type=jnp.float32)
            + b1_ref[...][None, :],
            0.0,
        )
        h = (
            jnp.dot(h, w2_ref[...], preferred_element_type=jnp.float32)
            + b2_ref[...][None, :]
        )
        if outer_relu:
            h = jnp.maximum(h, 0.0)
        o_ref[...] = h
        if want_sum:
            s_ref = rest[0]
            rows = i * B + lax.broadcasted_iota(jnp.int32, (B, 1), 0)
            hm = jnp.where(rows < n, h, 0.0)

            @pl.when(i == 0)
            def _():
                s_ref[...] = jnp.zeros((1, F), jnp.float32)

            s_ref[...] += hm.sum(axis=0, keepdims=True)

    out_shape = [jax.ShapeDtypeStruct((n, F), jnp.float32)]
    out_specs = [pl.BlockSpec((B, F), lambda i: (i, 0))]
    if want_sum:
        out_shape.append(jax.ShapeDtypeStruct((1, F), jnp.float32))
        out_specs.append(pl.BlockSpec((1, F), lambda i: (0, 0)))
    res = pl.pallas_call(
        body,
        grid=(grid,),
        in_specs=[
            pl.BlockSpec((B, F), lambda i: (i, 0)),
            pl.BlockSpec((NC, B, F), lambda i: (0, i + off_blocks, 0)),
            pl.BlockSpec((F, F), lambda i: (0, 0)),
            pl.BlockSpec((F,), lambda i: (0,)),
            pl.BlockSpec((F, F), lambda i: (0, 0)),
            pl.BlockSpec((F,), lambda i: (0,)),
        ],
        out_specs=out_specs,
        out_shape=out_shape,
    )(x, agg, w1, b1, w2, b2)
    return res if want_sum else res[0]


def _gat_pre(x, w, a_src, a_dst):
    n = x.shape[0]
    B = 1024
    grid = _ceil_div(n, B)
    neg = -3.0e38

    def body(x_ref, w_ref, as_ref, ad_ref, h_ref, s_ref, t_ref, sm_ref, tm_ref):
        i = pl.program_id(0)
        h = jnp.dot(x_ref[...], w_ref[...], preferred_element_type=jnp.float32)
        h_ref[...] = h
        s = jnp.dot(h, as_ref[...][:, None], preferred_element_type=jnp.float32)
        t = jnp.dot(h, ad_ref[...][:, None], preferred_element_type=jnp.float32)
        s_ref[...] = s
        t_ref[...] = t
        rows = i * B + lax.broadcasted_iota(jnp.int32, (B, 1), 0)
        valid = rows < n
        sm = jnp.max(jnp.where(valid, s, neg))
        tm = jnp.max(jnp.where(valid, t, neg))

        @pl.when(i == 0)
        def _():
            sm_ref[...] = jnp.full((L,), neg, jnp.float32)
            tm_ref[...] = jnp.full((L,), neg, jnp.float32)

        sm_ref[...] = jnp.maximum(sm_ref[...], sm)
        tm_ref[...] = jnp.maximum(tm_ref[...], tm)

    return pl.pallas_call(
        body,
        grid=(grid,),
        in_specs=[
            pl.BlockSpec((B, F), lambda i: (i, 0)),
            pl.BlockSpec((F, F), lambda i: (0, 0)),
            pl.BlockSpec((F,), lambda i: (0,)),
            pl.BlockSpec((F,), lambda i: (0,)),
        ],
        out_specs=[
            pl.BlockSpec((B, F), lambda i: (i, 0)),
            pl.BlockSpec((B, 1), lambda i: (i, 0)),
            pl.BlockSpec((B, 1), lambda i: (i, 0)),
            pl.BlockSpec((L,), lambda i: (0,)),
            pl.BlockSpec((L,), lambda i: (0,)),
        ],
        out_shape=[
            jax.ShapeDtypeStruct((n, F), jnp.float32),
            jax.ShapeDtypeStruct((n, 1), jnp.float32),
            jax.ShapeDtypeStruct((n, 1), jnp.float32),
            jax.ShapeDtypeStruct((L,), jnp.float32),
            jax.ShapeDtypeStruct((L,), jnp.float32),
        ],
    )(x, w, a_src, a_dst)


def _finalize(nump, denp, b, nq, ntot):
    B = 1024
    grid = _ceil_div(ntot, B)

    def body(n_ref, d_ref, b_ref, att_ref, qs_ref, ds_ref):
        i = pl.program_id(0)
        nsum = n_ref[...][0] + n_ref[...][1]
        den = d_ref[...][0] + d_ref[...][1] + 1e-16
        att = nsum / den[:, None] + b_ref[...][None, :]
        att_ref[...] = att
        rows = i * B + lax.broadcasted_iota(jnp.int32, (B, 1), 0)
        attv = jnp.where(rows < ntot, att, 0.0)
        qm = rows < nq

        @pl.when(i == 0)
        def _():
            qs_ref[...] = jnp.zeros((1, F), jnp.float32)
            ds_ref[...] = jnp.zeros((1, F), jnp.float32)

        qs_ref[...] += jnp.where(qm, attv, 0.0).sum(axis=0, keepdims=True)
        ds_ref[...] += jnp.where(qm, 0.0, attv).sum(axis=0, keepdims=True)

    return pl.pallas_call(
        body,
        grid=(grid,),
        in_specs=[
            pl.BlockSpec((NC, B, F), lambda i: (0, i, 0)),
            pl.BlockSpec((NC, B), lambda i: (0, i)),
            pl.BlockSpec((F,), lambda i: (0,)),
        ],
        out_specs=[
            pl.BlockSpec((B, F), lambda i: (i, 0)),
            pl.BlockSpec((1, F), lambda i: (0, 0)),
            pl.BlockSpec((1, F), lambda i: (0, 0)),
        ],
        out_shape=[
            jax.ShapeDtypeStruct((ntot, F), jnp.float32),
            jax.ShapeDtypeStruct((1, F), jnp.float32),
            jax.ShapeDtypeStruct((1, F), jnp.float32),
        ],
    )(nump, denp, b)


def _head(qa, qb, da, db, w1, b1, w2, b2, w3, b3, w4, b4):
    def body(qa_ref, qb_ref, da_ref, db_ref, w1_ref, b1_ref, w2_ref, b2_ref,
             w3_ref, b3_ref, w4_ref, b4_ref, o_ref):
        w1v = w1_ref[...]
        h = (
            jnp.dot(qa_ref[...], w1v[0:128], preferred_element_type=jnp.float32)
            + jnp.dot(qb_ref[...], w1v[128:256], preferred_element_type=jnp.float32)
            + jnp.dot(da_ref[...], w1v[256:384], preferred_element_type=jnp.float32)
            + jnp.dot(db_ref[...], w1v[384:512], preferred_element_type=jnp.float32)
            + b1_ref[...][None, :]
        )
        h = jnp.dot(h, w2_ref[...], preferred_element_type=jnp.float32) + b2_ref[...][None, :]
        h = jnp.maximum(h, 0.0)
        h = jnp.dot(h, w3_ref[...], preferred_element_type=jnp.float32) + b3_ref[...][None, :]
        h = jnp.maximum(h, 0.0)
        h = jnp.dot(h, w4_ref[...], preferred_element_type=jnp.float32) + b4_ref[...][None, :]
        o_ref[...] = jnp.maximum(h, 0.0)

    return pl.pallas_call(
        body,
        out_shape=jax.ShapeDtypeStruct((1, 1), jnp.float32),
    )(qa, qb, da, db, w1, b1, w2, b2, w3, b3, w4, b4)


def _prep_edges(src, dst, n_acc, ch):
    e = src.shape[0]
    unit = 2 * ch * NW
    epad = _ceil_div(e, unit) * unit
    npad = epad - e
    if npad:
        fill_src = (jnp.arange(npad, dtype=jnp.int32) % 64)
        fill_dst = n_acc + (jnp.arange(npad, dtype=jnp.int32) % 8)
        src = jnp.concatenate([src, fill_src])
        dst = jnp.concatenate([dst, fill_dst])
    per = epad // (NW * ch)
    src3 = src.reshape(NW, per, ch)
    dst3 = dst.reshape(NW, per, ch)
    eidx = jnp.stack([src3, dst3], axis=2)
    return eidx, per, npad


def kernel(query_in_feat, data_in_feat, query_edge_list, data_edge_list,
           query2data_edge_list, qg_W1, qg_b1, qg_W2, qg_b2, qg_W3, qg_b3,
           qg_W4, qg_b4, dg_W1, dg_b1, dg_W2, dg_b2, dg_W3, dg_b3, dg_W4,
           dg_b4, gat_W, gat_a_src, gat_a_dst, gat_b, L1_W, L1_b, L2_W, L2_b,
           L3_W, L3_b, L4_W, L4_b):
    nq = query_in_feat.shape[0]
    nd = data_in_feat.shape[0]
    ntot = nq + nd

    qe = query_edge_list.astype(jnp.int32)
    de = data_edge_list.astype(jnp.int32)
    xe = query2data_edge_list.astype(jnp.int32)

    csrc = jnp.concatenate([qe[0], de[0] + nq])
    cdst = jnp.concatenate([qe[1], de[1] + nq])
    cidx, cper, cpad = _prep_edges(csrc, cdst, ntot, CH_GIN)
    xidx, xper, xpad = _prep_edges(xe[0], xe[1], ntot, CH_GAT)

    segsum = _make_segsum(ntot + (8 if cpad else 0), ntot, cper, CH_GIN)
    gat_edges = _make_gat_edges(ntot + (8 if xpad else 0), ntot, xper, CH_GAT)

    x0 = jnp.concatenate([query_in_feat, data_in_feat], axis=0)

    agg1 = segsum(x0, cidx)
    hq = _mlp2(query_in_feat, agg1, qg_W1, qg_b1, qg_W2, qg_b2,
               off_blocks=0, outer_relu=True, want_sum=False)
    hd = _mlp2(data_in_feat, agg1, dg_W1, dg_b1, dg_W2, dg_b2,
               off_blocks=1, outer_relu=True, want_sum=False)

    x1 = jnp.concatenate([hq, hd], axis=0)
    agg2 = segsum(x1, cidx)
    query_x, qsA = _mlp2(hq, agg2, qg_W3, qg_b3, qg_W4, qg_b4,
                         off_blocks=0, outer_relu=False, want_sum=True)
    data_x, dsA = _mlp2(hd, agg2, dg_W3, dg_b3, dg_W4, dg_b4,
                        off_blocks=1, outer_relu=False, want_sum=True)

    hg, s, t, smax, tmax = _gat_pre(x0, gat_W, gat_a_src, gat_a_dst)
    nump, denp = gat_edges(hg, s[:, 0], t[:, 0], smax, tmax, xidx)
    att, qsB, dsB = _finalize(nump, denp.reshape(NC, ntot), gat_b, nq, ntot)

    pred = _head(qsA, qsB, dsA, dsB, L1_W, L1_b, L2_W, L2_b, L3_W, L3_b,
                 L4_W, L4_b)

    out_q = jnp.concatenate([query_x, att[:nq]], axis=1)
    out_d = jnp.concatenate([data_x, att[nq:]], axis=1)
    return (pred, out_q, out_d)
